# baseline (speedup 1.0000x reference)
import jax
import jax.numpy as jnp
from jax import lax
from jax.experimental import pallas as pl
from jax.experimental.pallas import tpu as pltpu
from jax.experimental.pallas import tpu_sc as plsc

N = 10000
E = 320000
NC = 2
NS = 16
NW = NC * NS
L = 16
C = 32
B = 128
NB = 4
CH = 160
CH2 = 80
EPT = CH * B
E_PAD = NS * EPT
N_PAD = 10240
RPT = N_PAD // NS

_F32 = jnp.float32
_I32 = jnp.int32

_GDN = lax.GatherDimensionNumbers(
    offset_dims=(), collapsed_slice_dims=(0,), start_index_map=(0,))


def _lane_splat(vec, i):
    idx = jnp.full((L, 1), i, _I32)
    return lax.gather(vec, idx, _GDN, slice_sizes=(1,),
                      mode=lax.GatherScatterMode.PROMISE_IN_BOUNDS)


def _mesh():
    return plsc.VectorSubcoreMesh(core_axis_name="c", subcore_axis_name="s")


_SC_PARAMS = pltpu.CompilerParams(use_tc_tiling_on_sc=False,
                                  needs_layout_passes=False)


def _zero_acc(s, acc, sbuf):
    def _z(i, _):
        sbuf[0, i, pl.ds(0, L)] = jnp.zeros((L,), _F32)
        sbuf[0, i, pl.ds(L, L)] = jnp.zeros((L,), _F32)
        return 0
    lax.fori_loop(0, B, _z, 0)
    for k in range(RPT // B):
        pltpu.sync_copy(sbuf.at[0], acc.at[pl.ds(s * RPT + k * B, B)])


def _scale32(rows, sbuf, nrm, b, j):
    for g in range(B // L):
        nv = nrm[j, pl.ds(g * L, L)]
        for i in range(L):
            e = g * L + i
            sp = _lane_splat(nv, i)
            sbuf[b, e, pl.ds(0, L)] = rows[b, e, pl.ds(0, L)] * sp
            sbuf[b, e, pl.ds(L, L)] = rows[b, e, pl.ds(L, L)] * sp


def _engine(issue, wait, idx_d, nrm, acc, rows, sbuf, ssem, chn):
    for b in range(NB):
        issue(b, b)

    def _ring(jo, _):
        for b in range(NB):
            j = jo * NB + b
            wait(j, b)

            @pl.when(jo > 0)
            def _(b=b, j=j):
                pltpu.make_async_copy(
                    sbuf.at[b], acc.at[idx_d.at[j - NB]], ssem.at[b]).wait()

            _scale32(rows, sbuf, nrm, b, j)

            @pl.when(j + NB < chn)
            def _(b=b, j=j):
                issue(j + NB, b)

            pltpu.async_copy(sbuf.at[b], acc.at[idx_d.at[j]],
                             ssem.at[b], add=True)
        return 0
    lax.fori_loop(0, chn // NB, _ring, 0)

    for b in range(NB):
        pltpu.make_async_copy(sbuf.at[b], acc.at[idx_d.at[chn - NB + b]],
                              ssem.at[b]).wait()


def _pair_body(ha, hb, srcp, dstp, nrmp, outa, outb,
               idx_s, idx_d, nrm, rows, sbuf, acc, gsem, ssem):
    c = lax.axis_index("c")
    s = lax.axis_index("s")

    pltpu.sync_copy(srcp.at[s], idx_s)
    pltpu.sync_copy(dstp.at[s], idx_d)
    pltpu.sync_copy(nrmp.at[s], nrm)
    _zero_acc(s, acc, sbuf)
    plsc.subcore_barrier()

    def issue(j, b):
        @pl.when(c == 0)
        def _():
            pltpu.async_copy(ha.at[idx_s.at[j]], rows.at[b], gsem.at[b])

        @pl.when(c == 1)
        def _():
            pltpu.async_copy(hb.at[idx_s.at[j]], rows.at[b], gsem.at[b])

    def wait(j, b):
        pltpu.make_async_copy(ha.at[idx_s.at[j]], rows.at[b],
                              gsem.at[b]).wait()

    _engine(issue, wait, idx_d, nrm, acc, rows, sbuf, ssem, CH)

    plsc.subcore_barrier()
    sl = pl.ds(s * RPT, RPT)

    @pl.when(c == 0)
    def _():
        pltpu.sync_copy(acc.at[sl], outa.at[sl])

    @pl.when(c == 1)
    def _():
        pltpu.sync_copy(acc.at[sl], outb.at[sl])


def _esplit_body(h, srcp, dstp, nrmp, out0, out1,
                 idx_s, idx_d, nrm, rows, sbuf, acc, gsem, ssem):
    c = lax.axis_index("c")
    s = lax.axis_index("s")
    w = c * NS + s

    pltpu.sync_copy(srcp.at[w], idx_s)
    pltpu.sync_copy(dstp.at[w], idx_d)
    pltpu.sync_copy(nrmp.at[w], nrm)
    _zero_acc(s, acc, sbuf)
    plsc.subcore_barrier()

    def issue(j, b):
        pltpu.async_copy(h.at[idx_s.at[j]], rows.at[b], gsem.at[b])

    def wait(j, b):
        pltpu.make_async_copy(h.at[idx_s.at[j]], rows.at[b],
                              gsem.at[b]).wait()

    _engine(issue, wait, idx_d, nrm, acc, rows, sbuf, ssem, CH2)

    plsc.subcore_barrier()
    sl = pl.ds(s * RPT, RPT)

    @pl.when(c == 0)
    def _():
        pltpu.sync_copy(acc.at[sl], out0.at[sl])

    @pl.when(c == 1)
    def _():
        pltpu.sync_copy(acc.at[sl], out1.at[sl])


_PROP_OUT = (jax.ShapeDtypeStruct((N_PAD, C), _F32),
             jax.ShapeDtypeStruct((N_PAD, C), _F32))


def _prop_scratch(chn):
    return [
        pltpu.VMEM((chn, B), _I32),
        pltpu.VMEM((chn, B), _I32),
        pltpu.VMEM((chn, B), _F32),
        pltpu.VMEM((NB, B, C), _F32),
        pltpu.VMEM((NB, B, C), _F32),
        pltpu.VMEM_SHARED((N_PAD, C), _F32),
        pltpu.SemaphoreType.DMA((NB,)),
        pltpu.SemaphoreType.DMA((NB,)),
    ]


@jax.jit
def _pair(ha, hb, srcp, dstp, nrmp):
    return pl.kernel(
        _pair_body,
        out_type=_PROP_OUT,
        mesh=_mesh(),
        scratch_types=_prop_scratch(CH),
        compiler_params=_SC_PARAMS,
    )(ha, hb, srcp, dstp, nrmp)


@jax.jit
def _esplit(h, srcp, dstp, nrmp):
    return pl.kernel(
        _esplit_body,
        out_type=_PROP_OUT,
        mesh=_mesh(),
        scratch_types=_prop_scratch(CH2),
        compiler_params=_SC_PARAMS,
    )(h, srcp, dstp, nrmp)


def _norm_body(srcp, dstp, wp, dis, nout, src_v, dst_v, w_v, dis_v, nrm_v):
    c = lax.axis_index("c")
    s = lax.axis_index("s")

    @pl.when(c == 0)
    def _():
        pltpu.sync_copy(srcp.at[s], src_v)
        pltpu.sync_copy(dstp.at[s], dst_v)
        pltpu.sync_copy(wp.at[s], w_v)
        pltpu.sync_copy(dis, dis_v)

        def _row(j, _):
            def _grp(g, _):
                sl = pl.ds(g * L, L)
                s16 = src_v[j, sl]
                d16 = dst_v[j, sl]
                w16 = w_v[j, sl]
                g1 = plsc.load_gather(dis_v, [s16])
                g2 = plsc.load_gather(dis_v, [d16])
                nrm_v[j, sl] = (0.0 - g1) * w16 * g2
                return 0
            lax.fori_loop(0, B // L, _grp, 0)
            return 0
        lax.fori_loop(0, CH, _row, 0)
        pltpu.sync_copy(nrm_v, nout.at[s])


@jax.jit
def _norm(srcp, dstp, wp, dis):
    return pl.kernel(
        _norm_body,
        out_type=jax.ShapeDtypeStruct((NS, CH, B), _F32),
        mesh=_mesh(),
        scratch_types=[
            pltpu.VMEM((CH, B), _I32),
            pltpu.VMEM((CH, B), _I32),
            pltpu.VMEM((CH, B), _F32),
            pltpu.VMEM((N,), _F32),
            pltpu.VMEM((CH, B), _F32),
        ],
        compiler_params=_SC_PARAMS,
    )(srcp, dstp, wp, dis)


def _instnorm_relu(y):
    mu = jnp.mean(y, axis=0, keepdims=True)
    var = jnp.mean((y - mu) ** 2, axis=0, keepdims=True)
    return jnp.maximum((y - mu) * lax.rsqrt(var + 1e-5), 0.0)


def _dot(a, b):
    return jnp.dot(a, b, preferred_element_type=_F32)


def _tc0_body(x_ref, d0_ref, d1_ref, w1_ref, dis_ref, u_ref, v_ref,
              base_ref):
    x = x_ref[...]
    deg = d0_ref[...][:N, 0:1] + d1_ref[...][:N, 0:1]
    dis_ref[...] = jnp.where(deg > 0, lax.rsqrt(deg), 0.0)
    W = w1_ref[...]
    v = _dot(x, W[2])
    u_ref[:N, :] = _dot(x, W[1])
    v_ref[:N, :] = v
    base_ref[...] = _dot(x, W[0]) - v


def _tc1_body(b1_ref, p_ref, r0_ref, r1_ref, w2_ref, h_ref, b2_ref):
    y = (b1_ref[...] + p_ref[...][:N]
         + 2.0 * (r0_ref[...][:N] + r1_ref[...][:N]))
    h = _instnorm_relu(y)
    h_ref[:N, :] = h
    W = w2_ref[...]
    b2_ref[...] = _dot(h, W[0] - W[2])


def _add2_body(a_ref, b_ref, o_ref):
    o_ref[...] = a_ref[...] + b_ref[...]


def _tc2_body(b2_ref, s1_ref, s2a_ref, s2b_ref, w2_ref, w3_ref, h_ref,
              b3_ref):
    W2 = w2_ref[...]
    y = (b2_ref[...]
         + _dot(s1_ref[...][:N], W2[1])
         + 2.0 * _dot(s2a_ref[...][:N] + s2b_ref[...][:N], W2[2]))
    h = _instnorm_relu(y)
    h_ref[:N, :] = h
    W3 = w3_ref[...]
    b3_ref[...] = _dot(h, W3[0] - W3[2])


def _tc3_body(b3_ref, t1_ref, t2a_ref, t2b_ref, w3_ref, x_ref, out_ref):
    W3 = w3_ref[...]
    y = (b3_ref[...]
         + _dot(t1_ref[...][:N], W3[1])
         + 2.0 * _dot(t2a_ref[...][:N] + t2b_ref[...][:N], W3[2]))
    out_ref[...] = _instnorm_relu(y) + x_ref[...]


def _tc_call(body, out_shapes, *args):
    return pl.pallas_call(
        body,
        out_shape=tuple(jax.ShapeDtypeStruct(s, _F32) for s in out_shapes),
    )(*args)


@jax.jit
def kernel(x, edge_index, edge_weight, W1, b1, W2, b2, W3, b3):
    x2 = x[0]
    pad = E_PAD - E
    src = jnp.pad(edge_index[0], (0, pad))
    dst = jnp.pad(edge_index[1], (0, pad))
    w = jnp.pad(edge_weight, (0, pad))
    srcp16 = src.reshape(NS, CH, B)
    dstp16 = dst.reshape(NS, CH, B)
    wp16 = w.reshape(NS, CH, B)
    srcp32 = src.reshape(NW, CH2, B)
    dstp32 = dst.reshape(NW, CH2, B)
    wp32 = w.reshape(NW, CH2, B)
    ones = jnp.ones((N_PAD, C), _F32)

    d0, d1 = _esplit(ones, srcp32, srcp32, wp32)

    dis, u, v, base1 = _tc_call(
        _tc0_body, ((N, 1), (N_PAD, C), (N_PAD, C), (N, C)),
        x2, d0, d1, W1)

    nrmp16 = _norm(srcp16, dstp16, wp16, dis.reshape(N))
    nrmp32 = nrmp16.reshape(NW, CH2, B)

    p, q = _pair(u, v, srcp16, dstp16, nrmp16)
    r0, r1 = _esplit(q, srcp32, dstp32, nrmp32)

    h1, base2 = _tc_call(
        _tc1_body, ((N_PAD, C), (N, C)), base1, p, r0, r1, W2)

    s1a, s1b = _esplit(h1, srcp32, dstp32, nrmp32)
    s1 = _tc_call(_add2_body, ((N_PAD, C),), s1a, s1b)[0]
    s2a, s2b = _esplit(s1, srcp32, dstp32, nrmp32)

    h2, base3 = _tc_call(
        _tc2_body, ((N_PAD, C), (N, 128)), base2, s1, s2a, s2b, W2, W3)

    t1a, t1b = _esplit(h2, srcp32, dstp32, nrmp32)
    t1 = _tc_call(_add2_body, ((N_PAD, C),), t1a, t1b)[0]
    t2a, t2b = _esplit(t1, srcp32, dstp32, nrmp32)

    y = _tc_call(
        _tc3_body, ((N, 128),), base3, t1, t2a, t2b, W3, x2)[0]

    return y[None]


# speedup vs baseline: 1.0622x; 1.0622x over previous
import jax
import jax.numpy as jnp
from jax import lax
from jax.experimental import pallas as pl
from jax.experimental.pallas import tpu as pltpu
from jax.experimental.pallas import tpu_sc as plsc

N = 10000
E = 320000
NC = 2
NS = 16
NW = NC * NS
L = 16
C = 32
B = 128
NB = 4
CH = 160
CH2 = 80
EPT = CH * B
E_PAD = NS * EPT
N_PAD = 10240
RPT = N_PAD // NS

_F32 = jnp.float32
_I32 = jnp.int32

_GDN = lax.GatherDimensionNumbers(
    offset_dims=(), collapsed_slice_dims=(0,), start_index_map=(0,))


def _lane_splat(vec, i):
    idx = jnp.full((L, 1), i, _I32)
    return lax.gather(vec, idx, _GDN, slice_sizes=(1,),
                      mode=lax.GatherScatterMode.PROMISE_IN_BOUNDS)


def _mesh():
    return plsc.VectorSubcoreMesh(core_axis_name="c", subcore_axis_name="s")


_SC_PARAMS = pltpu.CompilerParams(use_tc_tiling_on_sc=False,
                                  needs_layout_passes=False)


def _zero_acc(s, acc, sbuf):
    def _z(i, _):
        sbuf[0, i, pl.ds(0, L)] = jnp.zeros((L,), _F32)
        sbuf[0, i, pl.ds(L, L)] = jnp.zeros((L,), _F32)
        return 0
    lax.fori_loop(0, B, _z, 0)
    for k in range(RPT // B):
        pltpu.sync_copy(sbuf.at[0], acc.at[pl.ds(s * RPT + k * B, B)])


def _scale32(rows, sbuf, nrm, b, j):
    for g in range(B // L):
        nv = nrm[j, pl.ds(g * L, L)]
        for i in range(L):
            e = g * L + i
            sp = _lane_splat(nv, i)
            sbuf[b, e, pl.ds(0, L)] = rows[b, e, pl.ds(0, L)] * sp
            sbuf[b, e, pl.ds(L, L)] = rows[b, e, pl.ds(L, L)] * sp


def _engine(issue, wait, idx_d, nrm, acc, rows, sbuf, ssem, chn):
    for b in range(NB):
        issue(b, b)

    def _ring(jo, _):
        for b in range(NB):
            j = jo * NB + b
            wait(j, b)

            @pl.when(jo > 0)
            def _(b=b, j=j):
                pltpu.make_async_copy(
                    sbuf.at[b], acc.at[idx_d.at[j - NB]], ssem.at[b]).wait()

            _scale32(rows, sbuf, nrm, b, j)

            @pl.when(j + NB < chn)
            def _(b=b, j=j):
                issue(j + NB, b)

            pltpu.async_copy(sbuf.at[b], acc.at[idx_d.at[j]],
                             ssem.at[b], add=True)
        return 0
    lax.fori_loop(0, chn // NB, _ring, 0)

    for b in range(NB):
        pltpu.make_async_copy(sbuf.at[b], acc.at[idx_d.at[chn - NB + b]],
                              ssem.at[b]).wait()


def _pair_body(ha, hb, srcp, dstp, nrmp, outa, outb,
               idx_s, idx_d, nrm, rows, sbuf, acc, gsem, ssem):
    c = lax.axis_index("c")
    s = lax.axis_index("s")

    pltpu.sync_copy(srcp.at[s], idx_s)
    pltpu.sync_copy(dstp.at[s], idx_d)
    pltpu.sync_copy(nrmp.at[s], nrm)
    _zero_acc(s, acc, sbuf)
    plsc.subcore_barrier()

    def issue(j, b):
        @pl.when(c == 0)
        def _():
            pltpu.async_copy(ha.at[idx_s.at[j]], rows.at[b], gsem.at[b])

        @pl.when(c == 1)
        def _():
            pltpu.async_copy(hb.at[idx_s.at[j]], rows.at[b], gsem.at[b])

    def wait(j, b):
        pltpu.make_async_copy(ha.at[idx_s.at[j]], rows.at[b],
                              gsem.at[b]).wait()

    _engine(issue, wait, idx_d, nrm, acc, rows, sbuf, ssem, CH)

    plsc.subcore_barrier()
    sl = pl.ds(s * RPT, RPT)

    @pl.when(c == 0)
    def _():
        pltpu.sync_copy(acc.at[sl], outa.at[sl])

    @pl.when(c == 1)
    def _():
        pltpu.sync_copy(acc.at[sl], outb.at[sl])


def _esplit_body(h, srcp, dstp, nrmp, out0, out1,
                 idx_s, idx_d, nrm, rows, sbuf, acc, gsem, ssem):
    c = lax.axis_index("c")
    s = lax.axis_index("s")
    w = c * NS + s

    pltpu.sync_copy(srcp.at[w], idx_s)
    pltpu.sync_copy(dstp.at[w], idx_d)
    pltpu.sync_copy(nrmp.at[w], nrm)
    _zero_acc(s, acc, sbuf)
    plsc.subcore_barrier()

    def issue(j, b):
        pltpu.async_copy(h.at[idx_s.at[j]], rows.at[b], gsem.at[b])

    def wait(j, b):
        pltpu.make_async_copy(h.at[idx_s.at[j]], rows.at[b],
                              gsem.at[b]).wait()

    _engine(issue, wait, idx_d, nrm, acc, rows, sbuf, ssem, CH2)

    plsc.subcore_barrier()
    sl = pl.ds(s * RPT, RPT)

    @pl.when(c == 0)
    def _():
        pltpu.sync_copy(acc.at[sl], out0.at[sl])

    @pl.when(c == 1)
    def _():
        pltpu.sync_copy(acc.at[sl], out1.at[sl])


_PROP_OUT = (jax.ShapeDtypeStruct((N_PAD, C), _F32),
             jax.ShapeDtypeStruct((N_PAD, C), _F32))


def _prop_scratch(chn):
    return [
        pltpu.VMEM((chn, B), _I32),
        pltpu.VMEM((chn, B), _I32),
        pltpu.VMEM((chn, B), _F32),
        pltpu.VMEM((NB, B, C), _F32),
        pltpu.VMEM((NB, B, C), _F32),
        pltpu.VMEM_SHARED((N_PAD, C), _F32),
        pltpu.SemaphoreType.DMA((NB,)),
        pltpu.SemaphoreType.DMA((NB,)),
    ]


@jax.jit
def _pair(ha, hb, srcp, dstp, nrmp):
    return pl.kernel(
        _pair_body,
        out_type=_PROP_OUT,
        mesh=_mesh(),
        scratch_types=_prop_scratch(CH),
        compiler_params=_SC_PARAMS,
    )(ha, hb, srcp, dstp, nrmp)


@jax.jit
def _esplit(h, srcp, dstp, nrmp):
    return pl.kernel(
        _esplit_body,
        out_type=_PROP_OUT,
        mesh=_mesh(),
        scratch_types=_prop_scratch(CH2),
        compiler_params=_SC_PARAMS,
    )(h, srcp, dstp, nrmp)


def _deg_body(srcp, wp, out0, out1, idx_d, nrm, rows, sbuf, acc, gsem, ssem):
    c = lax.axis_index("c")
    s = lax.axis_index("s")
    w = c * NS + s

    pltpu.sync_copy(srcp.at[w], idx_d)
    pltpu.sync_copy(wp.at[w], nrm)
    _zero_acc(s, acc, sbuf)
    plsc.subcore_barrier()

    def _ring(jo, _):
        for b in range(NB):
            j = jo * NB + b

            @pl.when(jo > 0)
            def _(b=b, j=j):
                pltpu.make_async_copy(
                    sbuf.at[b], acc.at[idx_d.at[j - NB]], ssem.at[b]).wait()

            for g in range(B // L):
                nv = nrm[j, pl.ds(g * L, L)]
                for i in range(L):
                    e = g * L + i
                    sp = _lane_splat(nv, i)
                    sbuf[b, e, pl.ds(0, L)] = sp
                    sbuf[b, e, pl.ds(L, L)] = sp

            pltpu.async_copy(sbuf.at[b], acc.at[idx_d.at[j]],
                             ssem.at[b], add=True)
        return 0
    lax.fori_loop(0, CH2 // NB, _ring, 0)

    for b in range(NB):
        pltpu.make_async_copy(sbuf.at[b], acc.at[idx_d.at[CH2 - NB + b]],
                              ssem.at[b]).wait()

    plsc.subcore_barrier()
    sl = pl.ds(s * RPT, RPT)

    @pl.when(c == 0)
    def _():
        pltpu.sync_copy(acc.at[sl], out0.at[sl])

    @pl.when(c == 1)
    def _():
        pltpu.sync_copy(acc.at[sl], out1.at[sl])


@jax.jit
def _deg(srcp, wp):
    return pl.kernel(
        _deg_body,
        out_type=_PROP_OUT,
        mesh=_mesh(),
        scratch_types=[
            pltpu.VMEM((CH2, B), _I32),
            pltpu.VMEM((CH2, B), _F32),
            pltpu.VMEM((NB, B, C), _F32),
            pltpu.VMEM((NB, B, C), _F32),
            pltpu.VMEM_SHARED((N_PAD, C), _F32),
            pltpu.SemaphoreType.DMA((NB,)),
            pltpu.SemaphoreType.DMA((NB,)),
        ],
        compiler_params=_SC_PARAMS,
    )(srcp, wp)


def _norm_body(srcp, dstp, wp, dis, nout, src_v, dst_v, w_v, dis_v, nrm_v):
    c = lax.axis_index("c")
    s = lax.axis_index("s")

    @pl.when(c == 0)
    def _():
        pltpu.sync_copy(srcp.at[s], src_v)
        pltpu.sync_copy(dstp.at[s], dst_v)
        pltpu.sync_copy(wp.at[s], w_v)
        pltpu.sync_copy(dis, dis_v)

        def _row(j, _):
            def _grp(g, _):
                sl = pl.ds(g * L, L)
                s16 = src_v[j, sl]
                d16 = dst_v[j, sl]
                w16 = w_v[j, sl]
                g1 = plsc.load_gather(dis_v, [s16])
                g2 = plsc.load_gather(dis_v, [d16])
                nrm_v[j, sl] = (0.0 - g1) * w16 * g2
                return 0
            lax.fori_loop(0, B // L, _grp, 0)
            return 0
        lax.fori_loop(0, CH, _row, 0)
        pltpu.sync_copy(nrm_v, nout.at[s])


@jax.jit
def _norm(srcp, dstp, wp, dis):
    return pl.kernel(
        _norm_body,
        out_type=jax.ShapeDtypeStruct((NS, CH, B), _F32),
        mesh=_mesh(),
        scratch_types=[
            pltpu.VMEM((CH, B), _I32),
            pltpu.VMEM((CH, B), _I32),
            pltpu.VMEM((CH, B), _F32),
            pltpu.VMEM((N,), _F32),
            pltpu.VMEM((CH, B), _F32),
        ],
        compiler_params=_SC_PARAMS,
    )(srcp, dstp, wp, dis)


def _instnorm_relu(y):
    mu = jnp.mean(y, axis=0, keepdims=True)
    var = jnp.mean((y - mu) ** 2, axis=0, keepdims=True)
    return jnp.maximum((y - mu) * lax.rsqrt(var + 1e-5), 0.0)


def _dot(a, b):
    return jnp.dot(a, b, preferred_element_type=_F32)


def _tc0_body(x_ref, d0_ref, d1_ref, w1_ref, dis_ref, u_ref, v_ref,
              base_ref):
    x = x_ref[...]
    deg = d0_ref[...][:N, 0:1] + d1_ref[...][:N, 0:1]
    dis_ref[...] = jnp.where(deg > 0, lax.rsqrt(deg), 0.0)
    W = w1_ref[...]
    v = _dot(x, W[2])
    u_ref[:N, :] = _dot(x, W[1])
    v_ref[:N, :] = v
    base_ref[...] = _dot(x, W[0]) - v


def _tc1_body(b1_ref, p_ref, r0_ref, r1_ref, w2_ref, h_ref, b2_ref):
    y = (b1_ref[...] + p_ref[...][:N]
         + 2.0 * (r0_ref[...][:N] + r1_ref[...][:N]))
    h = _instnorm_relu(y)
    h_ref[:N, :] = h
    W = w2_ref[...]
    b2_ref[...] = _dot(h, W[0] - W[2])


def _add2_body(a_ref, b_ref, o_ref):
    o_ref[...] = a_ref[...] + b_ref[...]


def _tc2_body(b2_ref, s1_ref, s2a_ref, s2b_ref, w2_ref, w3_ref, h_ref,
              b3_ref):
    W2 = w2_ref[...]
    y = (b2_ref[...]
         + _dot(s1_ref[...][:N], W2[1])
         + 2.0 * _dot(s2a_ref[...][:N] + s2b_ref[...][:N], W2[2]))
    h = _instnorm_relu(y)
    h_ref[:N, :] = h
    W3 = w3_ref[...]
    b3_ref[...] = _dot(h, W3[0] - W3[2])


def _tc3_body(b3_ref, t1_ref, t2a_ref, t2b_ref, w3_ref, x_ref, out_ref):
    W3 = w3_ref[...]
    y = (b3_ref[...]
         + _dot(t1_ref[...][:N], W3[1])
         + 2.0 * _dot(t2a_ref[...][:N] + t2b_ref[...][:N], W3[2]))
    out_ref[...] = _instnorm_relu(y) + x_ref[...]


def _tc_call(body, out_shapes, *args):
    return pl.pallas_call(
        body,
        out_shape=tuple(jax.ShapeDtypeStruct(s, _F32) for s in out_shapes),
    )(*args)


@jax.jit
def kernel(x, edge_index, edge_weight, W1, b1, W2, b2, W3, b3):
    x2 = x[0]
    pad = E_PAD - E
    src = jnp.pad(edge_index[0], (0, pad))
    dst = jnp.pad(edge_index[1], (0, pad))
    w = jnp.pad(edge_weight, (0, pad))
    srcp16 = src.reshape(NS, CH, B)
    dstp16 = dst.reshape(NS, CH, B)
    wp16 = w.reshape(NS, CH, B)
    srcp32 = src.reshape(NW, CH2, B)
    dstp32 = dst.reshape(NW, CH2, B)
    wp32 = w.reshape(NW, CH2, B)
    d0, d1 = _deg(srcp32, wp32)

    dis, u, v, base1 = _tc_call(
        _tc0_body, ((N, 1), (N_PAD, C), (N_PAD, C), (N, C)),
        x2, d0, d1, W1)

    nrmp16 = _norm(srcp16, dstp16, wp16, dis.reshape(N))
    nrmp32 = nrmp16.reshape(NW, CH2, B)

    p, q = _pair(u, v, srcp16, dstp16, nrmp16)
    r0, r1 = _esplit(q, srcp32, dstp32, nrmp32)

    h1, base2 = _tc_call(
        _tc1_body, ((N_PAD, C), (N, C)), base1, p, r0, r1, W2)

    s1a, s1b = _esplit(h1, srcp32, dstp32, nrmp32)
    s1 = _tc_call(_add2_body, ((N_PAD, C),), s1a, s1b)[0]
    s2a, s2b = _esplit(s1, srcp32, dstp32, nrmp32)

    h2, base3 = _tc_call(
        _tc2_body, ((N_PAD, C), (N, 128)), base2, s1, s2a, s2b, W2, W3)

    t1a, t1b = _esplit(h2, srcp32, dstp32, nrmp32)
    t1 = _tc_call(_add2_body, ((N_PAD, C),), t1a, t1b)[0]
    t2a, t2b = _esplit(t1, srcp32, dstp32, nrmp32)

    y = _tc_call(
        _tc3_body, ((N, 128),), base3, t1, t2a, t2b, W3, x2)[0]

    return y[None]


# speedup vs baseline: 1.0677x; 1.0053x over previous
import jax
import jax.numpy as jnp
from jax import lax
from jax.experimental import pallas as pl
from jax.experimental.pallas import tpu as pltpu
from jax.experimental.pallas import tpu_sc as plsc

N = 10000
E = 320000
NC = 2
NS = 16
NW = NC * NS
L = 16
C = 32
B = 128
NB = 4
NB2 = 8
CH = 160
CH2 = 80
EPT = CH * B
E_PAD = NS * EPT
N_PAD = 10240
RPT = N_PAD // NS

_F32 = jnp.float32
_I32 = jnp.int32

_GDN = lax.GatherDimensionNumbers(
    offset_dims=(), collapsed_slice_dims=(0,), start_index_map=(0,))


def _lane_splat(vec, i):
    idx = jnp.full((L, 1), i, _I32)
    return lax.gather(vec, idx, _GDN, slice_sizes=(1,),
                      mode=lax.GatherScatterMode.PROMISE_IN_BOUNDS)


def _mesh():
    return plsc.VectorSubcoreMesh(core_axis_name="c", subcore_axis_name="s")


_SC_PARAMS = pltpu.CompilerParams(use_tc_tiling_on_sc=False,
                                  needs_layout_passes=False)


def _zero_acc(s, acc, sbuf):
    def _z(i, _):
        sbuf[0, i, pl.ds(0, L)] = jnp.zeros((L,), _F32)
        sbuf[0, i, pl.ds(L, L)] = jnp.zeros((L,), _F32)
        return 0
    lax.fori_loop(0, B, _z, 0)
    for k in range(RPT // B):
        pltpu.sync_copy(sbuf.at[0], acc.at[pl.ds(s * RPT + k * B, B)])


def _scale32(rows, sbuf, nrm, b, j):
    for g in range(B // L):
        nv = nrm[j, pl.ds(g * L, L)]
        for i in range(L):
            e = g * L + i
            sp = _lane_splat(nv, i)
            sbuf[b, e, pl.ds(0, L)] = rows[b, e, pl.ds(0, L)] * sp
            sbuf[b, e, pl.ds(L, L)] = rows[b, e, pl.ds(L, L)] * sp


def _engine(issue, wait, idx_d, nrm, acc, rows, sbuf, ssem, chn, nb):
    for b in range(nb):
        issue(b, b)

    def _ring(jo, _):
        for b in range(nb):
            j = jo * nb + b
            wait(j, b)

            @pl.when(jo > 0)
            def _(b=b, j=j):
                pltpu.make_async_copy(
                    sbuf.at[b], acc.at[idx_d.at[j - nb]], ssem.at[b]).wait()

            _scale32(rows, sbuf, nrm, b, j)

            @pl.when(j + nb < chn)
            def _(b=b, j=j):
                issue(j + nb, b)

            pltpu.async_copy(sbuf.at[b], acc.at[idx_d.at[j]],
                             ssem.at[b], add=True)
        return 0
    lax.fori_loop(0, chn // nb, _ring, 0)

    for b in range(nb):
        pltpu.make_async_copy(sbuf.at[b], acc.at[idx_d.at[chn - nb + b]],
                              ssem.at[b]).wait()


def _pair_body(ha, hb, srcp, dstp, nrmp, outa, outb,
               idx_s, idx_d, nrm, rows, sbuf, acc, gsem, ssem):
    c = lax.axis_index("c")
    s = lax.axis_index("s")

    pltpu.sync_copy(srcp.at[s], idx_s)
    pltpu.sync_copy(dstp.at[s], idx_d)
    pltpu.sync_copy(nrmp.at[s], nrm)
    _zero_acc(s, acc, sbuf)
    plsc.subcore_barrier()

    def issue(j, b):
        @pl.when(c == 0)
        def _():
            pltpu.async_copy(ha.at[idx_s.at[j]], rows.at[b], gsem.at[b])

        @pl.when(c == 1)
        def _():
            pltpu.async_copy(hb.at[idx_s.at[j]], rows.at[b], gsem.at[b])

    def wait(j, b):
        pltpu.make_async_copy(ha.at[idx_s.at[j]], rows.at[b],
                              gsem.at[b]).wait()

    _engine(issue, wait, idx_d, nrm, acc, rows, sbuf, ssem, CH, NB)

    plsc.subcore_barrier()
    sl = pl.ds(s * RPT, RPT)

    @pl.when(c == 0)
    def _():
        pltpu.sync_copy(acc.at[sl], outa.at[sl])

    @pl.when(c == 1)
    def _():
        pltpu.sync_copy(acc.at[sl], outb.at[sl])


def _esplit_body(h, srcp, dstp, nrmp, out0, out1,
                 idx_s, idx_d, nrm, rows, sbuf, acc, gsem, ssem):
    c = lax.axis_index("c")
    s = lax.axis_index("s")
    w = c * NS + s

    pltpu.sync_copy(srcp.at[w], idx_s)
    pltpu.sync_copy(dstp.at[w], idx_d)
    pltpu.sync_copy(nrmp.at[w], nrm)
    _zero_acc(s, acc, sbuf)
    plsc.subcore_barrier()

    def issue(j, b):
        pltpu.async_copy(h.at[idx_s.at[j]], rows.at[b], gsem.at[b])

    def wait(j, b):
        pltpu.make_async_copy(h.at[idx_s.at[j]], rows.at[b],
                              gsem.at[b]).wait()

    _engine(issue, wait, idx_d, nrm, acc, rows, sbuf, ssem, CH2, NB2)

    plsc.subcore_barrier()
    sl = pl.ds(s * RPT, RPT)

    @pl.when(c == 0)
    def _():
        pltpu.sync_copy(acc.at[sl], out0.at[sl])

    @pl.when(c == 1)
    def _():
        pltpu.sync_copy(acc.at[sl], out1.at[sl])


_PROP_OUT = (jax.ShapeDtypeStruct((N_PAD, C), _F32),
             jax.ShapeDtypeStruct((N_PAD, C), _F32))


def _prop_scratch(chn, nb):
    return [
        pltpu.VMEM((chn, B), _I32),
        pltpu.VMEM((chn, B), _I32),
        pltpu.VMEM((chn, B), _F32),
        pltpu.VMEM((nb, B, C), _F32),
        pltpu.VMEM((nb, B, C), _F32),
        pltpu.VMEM_SHARED((N_PAD, C), _F32),
        pltpu.SemaphoreType.DMA((nb,)),
        pltpu.SemaphoreType.DMA((nb,)),
    ]


@jax.jit
def _pair(ha, hb, srcp, dstp, nrmp):
    return pl.kernel(
        _pair_body,
        out_type=_PROP_OUT,
        mesh=_mesh(),
        scratch_types=_prop_scratch(CH, NB),
        compiler_params=_SC_PARAMS,
    )(ha, hb, srcp, dstp, nrmp)


@jax.jit
def _esplit(h, srcp, dstp, nrmp):
    return pl.kernel(
        _esplit_body,
        out_type=_PROP_OUT,
        mesh=_mesh(),
        scratch_types=_prop_scratch(CH2, NB2),
        compiler_params=_SC_PARAMS,
    )(h, srcp, dstp, nrmp)


def _deg_body(srcp, wp, out0, out1, idx_d, nrm, rows, sbuf, acc, gsem, ssem):
    c = lax.axis_index("c")
    s = lax.axis_index("s")
    w = c * NS + s

    pltpu.sync_copy(srcp.at[w], idx_d)
    pltpu.sync_copy(wp.at[w], nrm)
    _zero_acc(s, acc, sbuf)
    plsc.subcore_barrier()

    def _ring(jo, _):
        for b in range(NB2):
            j = jo * NB2 + b

            @pl.when(jo > 0)
            def _(b=b, j=j):
                pltpu.make_async_copy(
                    sbuf.at[b], acc.at[idx_d.at[j - NB2]], ssem.at[b]).wait()

            for g in range(B // L):
                nv = nrm[j, pl.ds(g * L, L)]
                for i in range(L):
                    e = g * L + i
                    sp = _lane_splat(nv, i)
                    sbuf[b, e, pl.ds(0, L)] = sp
                    sbuf[b, e, pl.ds(L, L)] = sp

            pltpu.async_copy(sbuf.at[b], acc.at[idx_d.at[j]],
                             ssem.at[b], add=True)
        return 0
    lax.fori_loop(0, CH2 // NB2, _ring, 0)

    for b in range(NB2):
        pltpu.make_async_copy(sbuf.at[b], acc.at[idx_d.at[CH2 - NB2 + b]],
                              ssem.at[b]).wait()

    plsc.subcore_barrier()
    sl = pl.ds(s * RPT, RPT)

    @pl.when(c == 0)
    def _():
        pltpu.sync_copy(acc.at[sl], out0.at[sl])

    @pl.when(c == 1)
    def _():
        pltpu.sync_copy(acc.at[sl], out1.at[sl])


@jax.jit
def _deg(srcp, wp):
    return pl.kernel(
        _deg_body,
        out_type=_PROP_OUT,
        mesh=_mesh(),
        scratch_types=[
            pltpu.VMEM((CH2, B), _I32),
            pltpu.VMEM((CH2, B), _F32),
            pltpu.VMEM((NB2, B, C), _F32),
            pltpu.VMEM((NB2, B, C), _F32),
            pltpu.VMEM_SHARED((N_PAD, C), _F32),
            pltpu.SemaphoreType.DMA((NB2,)),
            pltpu.SemaphoreType.DMA((NB2,)),
        ],
        compiler_params=_SC_PARAMS,
    )(srcp, wp)


def _norm_body(srcp, dstp, wp, dis, nout, src_v, dst_v, w_v, dis_v, nrm_v):
    c = lax.axis_index("c")
    s = lax.axis_index("s")

    @pl.when(c == 0)
    def _():
        pltpu.sync_copy(srcp.at[s], src_v)
        pltpu.sync_copy(dstp.at[s], dst_v)
        pltpu.sync_copy(wp.at[s], w_v)
        pltpu.sync_copy(dis, dis_v)

        def _row(j, _):
            def _grp(g, _):
                sl = pl.ds(g * L, L)
                s16 = src_v[j, sl]
                d16 = dst_v[j, sl]
                w16 = w_v[j, sl]
                g1 = plsc.load_gather(dis_v, [s16])
                g2 = plsc.load_gather(dis_v, [d16])
                nrm_v[j, sl] = (0.0 - g1) * w16 * g2
                return 0
            lax.fori_loop(0, B // L, _grp, 0)
            return 0
        lax.fori_loop(0, CH, _row, 0)
        pltpu.sync_copy(nrm_v, nout.at[s])


@jax.jit
def _norm(srcp, dstp, wp, dis):
    return pl.kernel(
        _norm_body,
        out_type=jax.ShapeDtypeStruct((NS, CH, B), _F32),
        mesh=_mesh(),
        scratch_types=[
            pltpu.VMEM((CH, B), _I32),
            pltpu.VMEM((CH, B), _I32),
            pltpu.VMEM((CH, B), _F32),
            pltpu.VMEM((N,), _F32),
            pltpu.VMEM((CH, B), _F32),
        ],
        compiler_params=_SC_PARAMS,
    )(srcp, dstp, wp, dis)


def _instnorm_relu(y):
    mu = jnp.mean(y, axis=0, keepdims=True)
    var = jnp.mean((y - mu) ** 2, axis=0, keepdims=True)
    return jnp.maximum((y - mu) * lax.rsqrt(var + 1e-5), 0.0)


def _dot(a, b):
    return jnp.dot(a, b, preferred_element_type=_F32)


def _tc0_body(x_ref, d0_ref, d1_ref, w1_ref, dis_ref, u_ref, v_ref,
              base_ref):
    x = x_ref[...]
    deg = d0_ref[...][:N, 0:1] + d1_ref[...][:N, 0:1]
    dis_ref[...] = jnp.where(deg > 0, lax.rsqrt(deg), 0.0)
    W = w1_ref[...]
    v = _dot(x, W[2])
    u_ref[:N, :] = _dot(x, W[1])
    v_ref[:N, :] = v
    base_ref[...] = _dot(x, W[0]) - v


def _tc1_body(b1_ref, p_ref, r0_ref, r1_ref, w2_ref, h_ref, b2_ref):
    y = (b1_ref[...] + p_ref[...][:N]
         + 2.0 * (r0_ref[...][:N] + r1_ref[...][:N]))
    h = _instnorm_relu(y)
    h_ref[:N, :] = h
    W = w2_ref[...]
    b2_ref[...] = _dot(h, W[0] - W[2])


def _add2_body(a_ref, b_ref, o_ref):
    o_ref[...] = a_ref[...] + b_ref[...]


def _tc2_body(b2_ref, s1_ref, s2a_ref, s2b_ref, w2_ref, w3_ref, h_ref,
              b3_ref):
    W2 = w2_ref[...]
    y = (b2_ref[...]
         + _dot(s1_ref[...][:N], W2[1])
         + 2.0 * _dot(s2a_ref[...][:N] + s2b_ref[...][:N], W2[2]))
    h = _instnorm_relu(y)
    h_ref[:N, :] = h
    W3 = w3_ref[...]
    b3_ref[...] = _dot(h, W3[0] - W3[2])


def _tc3_body(b3_ref, t1_ref, t2a_ref, t2b_ref, w3_ref, x_ref, out_ref):
    W3 = w3_ref[...]
    y = (b3_ref[...]
         + _dot(t1_ref[...][:N], W3[1])
         + 2.0 * _dot(t2a_ref[...][:N] + t2b_ref[...][:N], W3[2]))
    out_ref[...] = _instnorm_relu(y) + x_ref[...]


def _tc_call(body, out_shapes, *args):
    return pl.pallas_call(
        body,
        out_shape=tuple(jax.ShapeDtypeStruct(s, _F32) for s in out_shapes),
    )(*args)


@jax.jit
def kernel(x, edge_index, edge_weight, W1, b1, W2, b2, W3, b3):
    x2 = x[0]
    pad = E_PAD - E
    src = jnp.pad(edge_index[0], (0, pad))
    dst = jnp.pad(edge_index[1], (0, pad))
    w = jnp.pad(edge_weight, (0, pad))
    srcp16 = src.reshape(NS, CH, B)
    dstp16 = dst.reshape(NS, CH, B)
    wp16 = w.reshape(NS, CH, B)
    srcp32 = src.reshape(NW, CH2, B)
    dstp32 = dst.reshape(NW, CH2, B)
    wp32 = w.reshape(NW, CH2, B)
    d0, d1 = _deg(srcp32, wp32)

    dis, u, v, base1 = _tc_call(
        _tc0_body, ((N, 1), (N_PAD, C), (N_PAD, C), (N, C)),
        x2, d0, d1, W1)

    nrmp16 = _norm(srcp16, dstp16, wp16, dis.reshape(N))
    nrmp32 = nrmp16.reshape(NW, CH2, B)

    p, q = _pair(u, v, srcp16, dstp16, nrmp16)
    r0, r1 = _esplit(q, srcp32, dstp32, nrmp32)

    h1, base2 = _tc_call(
        _tc1_body, ((N_PAD, C), (N, C)), base1, p, r0, r1, W2)

    s1a, s1b = _esplit(h1, srcp32, dstp32, nrmp32)
    s1 = _tc_call(_add2_body, ((N_PAD, C),), s1a, s1b)[0]
    s2a, s2b = _esplit(s1, srcp32, dstp32, nrmp32)

    h2, base3 = _tc_call(
        _tc2_body, ((N_PAD, C), (N, 128)), base2, s1, s2a, s2b, W2, W3)

    t1a, t1b = _esplit(h2, srcp32, dstp32, nrmp32)
    t1 = _tc_call(_add2_body, ((N_PAD, C),), t1a, t1b)[0]
    t2a, t2b = _esplit(t1, srcp32, dstp32, nrmp32)

    y = _tc_call(
        _tc3_body, ((N, 128),), base3, t1, t2a, t2b, W3, x2)[0]

    return y[None]
